# Initial kernel scaffold; baseline (speedup 1.0000x reference)
import jax
import jax.numpy as jnp
from jax.experimental import pallas as pl


def kernel(h, edge_index, W0, attn0, b0, W1, attn1, b1):
    raise NotImplementedError("write your pallas kernel here")


# speedup vs baseline: 16.2474x; 16.2474x over previous
import functools

import numpy as np
import jax
import jax.numpy as jnp
from jax import lax
from jax.experimental import pallas as pl
from jax.experimental.pallas import tpu as pltpu
from jax.experimental.pallas import tpu_sc as plsc

N = 10000
E = 320000
D = 128
WPAD = 144
NC, NS = 2, 16
NW = NC * NS
EPW = E // NW
GB = 80
NCHUNK = EPW // GB
RPW = N // NS
ZB = 25
NEG = 0.2

_MESH = plsc.VectorSubcoreMesh(core_axis_name="c", subcore_axis_name="s")
_SC_PARAMS = pltpu.CompilerParams(use_tc_tiling_on_sc=False)


def _mm_body(x_ref, w_ref, o_ref):
    o_ref[...] = jnp.dot(x_ref[...], w_ref[...],
                         preferred_element_type=jnp.float32)


def _matmul(x, w, bn=2000):
    n, k = x.shape
    m = w.shape[1]
    return pl.pallas_call(
        _mm_body,
        grid=(n // bn,),
        in_specs=[pl.BlockSpec((bn, k), lambda i: (i, 0)),
                  pl.BlockSpec((k, m), lambda i: (0, 0))],
        out_specs=pl.BlockSpec((bn, m), lambda i: (i, 0)),
        out_shape=jax.ShapeDtypeStruct((n, m), jnp.float32),
    )(x, w)


def _gather_body(feat, src, dst, fs, fd, idx_v, rows_v, sem):
    c = lax.axis_index("c")
    s = lax.axis_index("s")
    base0 = (c * NS + s) * EPW

    def step(i, carry):
        base = base0 + i * GB
        pltpu.sync_copy(src.at[pl.ds(base, GB)], idx_v)
        pltpu.async_copy(feat.at[idx_v], rows_v, sem).wait()
        pltpu.sync_copy(rows_v, fs.at[pl.ds(base, GB)])
        pltpu.sync_copy(dst.at[pl.ds(base, GB)], idx_v)
        pltpu.async_copy(feat.at[idx_v], rows_v, sem).wait()
        pltpu.sync_copy(rows_v, fd.at[pl.ds(base, GB)])
        return carry

    lax.fori_loop(0, NCHUNK, step, 0)


_gather = pl.kernel(
    _gather_body,
    out_type=(jax.ShapeDtypeStruct((E, D), jnp.float32),
              jax.ShapeDtypeStruct((E, D), jnp.float32)),
    mesh=_MESH,
    scratch_types=[pltpu.VMEM((GB,), jnp.int32),
                   pltpu.VMEM((GB, D), jnp.float32),
                   pltpu.SemaphoreType.DMA],
    compiler_params=_SC_PARAMS,
)


def _edge_body(fs_ref, fd_ref, attn_ref, s_ref, t_ref, o_ref):
    fs = fs_ref[...]
    e = fs + fd_ref[...]
    lr = jnp.where(e >= 0, e, NEG * e)
    w = lr * attn_ref[0][None, :]
    S = s_ref[...]
    z = jnp.dot(w, S, preferred_element_type=jnp.float32)
    p = jnp.exp(z)
    p128 = jnp.dot(p, S.T, preferred_element_type=jnp.float32)
    msg = fs * p128
    p16 = jnp.dot(p, t_ref[...], preferred_element_type=jnp.float32)
    o_ref[...] = jnp.concatenate([msg, p16], axis=1)


def _edge(fs, fd, attn_full, S, T, be=2000):
    return pl.pallas_call(
        _edge_body,
        grid=(E // be,),
        in_specs=[pl.BlockSpec((be, D), lambda i: (i, 0)),
                  pl.BlockSpec((be, D), lambda i: (i, 0)),
                  pl.BlockSpec((8, D), lambda i: (0, 0)),
                  pl.BlockSpec((D, 8), lambda i: (0, 0)),
                  pl.BlockSpec((8, 16), lambda i: (0, 0))],
        out_specs=pl.BlockSpec((be, WPAD), lambda i: (i, 0)),
        out_shape=jax.ShapeDtypeStruct((E, WPAD), jnp.float32),
    )(fs, fd, attn_full, S, T)


def _scatter_body(msgp, dst, out, idx_v, rows_v, zrow_v, sem, acc):
    c = lax.axis_index("c")
    s = lax.axis_index("s")
    wid = c * NS + s

    for r in range(ZB):
        for cc in range(WPAD // 16):
            zrow_v[r, pl.ds(cc * 16, 16)] = jnp.zeros((16,), jnp.float32)

    def zstep(j, carry):
        pltpu.sync_copy(zrow_v, acc.at[pl.ds(s * RPW + j * ZB, ZB)])
        return carry

    lax.fori_loop(0, RPW // ZB, zstep, 0)
    plsc.subcore_barrier()

    base0 = wid * EPW

    def step(i, carry):
        base = base0 + i * GB
        pltpu.sync_copy(dst.at[pl.ds(base, GB)], idx_v)
        pltpu.sync_copy(msgp.at[pl.ds(base, GB)], rows_v)
        pltpu.sync_copy(rows_v, acc.at[idx_v], add=True)
        return carry

    lax.fori_loop(0, NCHUNK, step, 0)
    plsc.subcore_barrier()
    pltpu.sync_copy(acc.at[pl.ds(s * RPW, RPW)],
                    out.at[c, pl.ds(s * RPW, RPW)])


_scatter = pl.kernel(
    _scatter_body,
    out_type=jax.ShapeDtypeStruct((NC, N, WPAD), jnp.float32),
    mesh=_MESH,
    scratch_types=[pltpu.VMEM((GB,), jnp.int32),
                   pltpu.VMEM((GB, WPAD), jnp.float32),
                   pltpu.VMEM((ZB, WPAD), jnp.float32),
                   pltpu.SemaphoreType.DMA,
                   pltpu.VMEM_SHARED((N, WPAD), jnp.float32)],
    compiler_params=_SC_PARAMS,
)


def _fin_mm_body(acc_ref, emat_ref, b_ref, w_ref, o_ref):
    a = acc_ref[0] + acc_ref[1]
    den = jnp.dot(a, emat_ref[...], preferred_element_type=jnp.float32)
    o = a[:, :D] / (den + 1e-9) + b_ref[0][None, :]
    o = jnp.where(o > 0, o, jnp.exp(o) - 1.0)
    o_ref[...] = jnp.dot(o, w_ref[...], preferred_element_type=jnp.float32)


def _fin_body(acc_ref, emat_ref, b_ref, o_ref):
    a = acc_ref[0] + acc_ref[1]
    den = jnp.dot(a, emat_ref[...], preferred_element_type=jnp.float32)
    o_ref[...] = a[:, :D] / (den + 1e-9) + b_ref[0][None, :]


def _finalize(acc, emat, b8, w=None, bn=2000):
    in_specs = [pl.BlockSpec((NC, bn, WPAD), lambda i: (0, i, 0)),
                pl.BlockSpec((WPAD, D), lambda i: (0, 0)),
                pl.BlockSpec((8, D), lambda i: (0, 0))]
    args = [acc, emat, b8]
    body = _fin_body
    if w is not None:
        in_specs.append(pl.BlockSpec((D, D), lambda i: (0, 0)))
        args.append(w)
        body = _fin_mm_body
    return pl.pallas_call(
        body,
        grid=(N // bn,),
        in_specs=in_specs,
        out_specs=pl.BlockSpec((bn, D), lambda i: (i, 0)),
        out_shape=jax.ShapeDtypeStruct((N, D), jnp.float32),
    )(*args)


def _head_maps(heads, hid):
    S = np.zeros((D, 8), np.float32)
    for h in range(heads):
        S[h * hid:(h + 1) * hid, h] = 1.0
    T = np.zeros((8, 16), np.float32)
    for h in range(heads):
        T[h, h] = 1.0
    emat = np.zeros((WPAD, D), np.float32)
    for h in range(heads):
        emat[D + h, h * hid:(h + 1) * hid] = 1.0
    return S, T, emat


_S0, _T0, _E0 = _head_maps(4, 32)
_S1, _T1, _E1 = _head_maps(1, 128)


def _bcast8(v):
    return jnp.broadcast_to(v.reshape(1, D), (8, D))


def kernel(h, edge_index, W0, attn0, b0, W1, attn1, b1):
    src = edge_index[0]
    dst = edge_index[1]

    def layer(feat, attn, S, T, emat, b, w_next):
        fs, fd = _gather(feat, src, dst)
        msgp = _edge(fs, fd, _bcast8(attn.reshape(-1)),
                     jnp.asarray(S), jnp.asarray(T))
        acc = _scatter(msgp, dst)
        return _finalize(acc, jnp.asarray(emat), _bcast8(b), w_next)

    feat0 = _matmul(h, W0)
    feat1 = layer(feat0, attn0, _S0, _T0, _E0, b0, W1)
    out = layer(feat1, attn1, _S1, _T1, _E1, b1, None)
    return out


# speedup vs baseline: 20.5819x; 1.2668x over previous
import functools

import numpy as np
import jax
import jax.numpy as jnp
from jax import lax
from jax.experimental import pallas as pl
from jax.experimental.pallas import tpu as pltpu
from jax.experimental.pallas import tpu_sc as plsc

N = 10000
E = 320000
D = 128
WPAD = 144
NC, NS = 2, 16
NW = NC * NS
EPW = E // NW
GB = 80
NCHUNK = EPW // GB
RPW = N // NS
ZB = 25
NEG = 0.2

_MESH = plsc.VectorSubcoreMesh(core_axis_name="c", subcore_axis_name="s")
_SC_PARAMS = pltpu.CompilerParams(use_tc_tiling_on_sc=False)


def _mm_body(x_ref, w_ref, o_ref):
    o_ref[...] = jnp.dot(x_ref[...], w_ref[...],
                         preferred_element_type=jnp.float32)


def _matmul(x, w, bn=2000):
    n, k = x.shape
    m = w.shape[1]
    return pl.pallas_call(
        _mm_body,
        grid=(n // bn,),
        in_specs=[pl.BlockSpec((bn, k), lambda i: (i, 0)),
                  pl.BlockSpec((k, m), lambda i: (0, 0))],
        out_specs=pl.BlockSpec((bn, m), lambda i: (i, 0)),
        out_shape=jax.ShapeDtypeStruct((n, m), jnp.float32),
    )(x, w)


def _gather_body(feat, src, dst, fs, fd, idx_v, rows_v, gsem, wsem):
    c = lax.axis_index("c")
    s = lax.axis_index("s")
    base0 = (c * NS + s) * EPW

    def start(i):
        slot = lax.rem(i, 2)
        base = base0 + i * GB
        pltpu.sync_copy(src.at[pl.ds(base, GB)], idx_v.at[slot, 0])
        pltpu.sync_copy(dst.at[pl.ds(base, GB)], idx_v.at[slot, 1])
        pltpu.async_copy(feat.at[idx_v.at[slot, 0]], rows_v.at[slot, 0], gsem)
        pltpu.async_copy(feat.at[idx_v.at[slot, 1]], rows_v.at[slot, 1], gsem)

    def gwait(i):
        slot = lax.rem(i, 2)
        pltpu.make_async_copy(feat.at[idx_v.at[slot, 0]],
                              rows_v.at[slot, 0], gsem).wait()
        pltpu.make_async_copy(feat.at[idx_v.at[slot, 1]],
                              rows_v.at[slot, 1], gsem).wait()

    def wstart(i):
        slot = lax.rem(i, 2)
        base = base0 + i * GB
        pltpu.async_copy(rows_v.at[slot, 0], fs.at[pl.ds(base, GB)], wsem)
        pltpu.async_copy(rows_v.at[slot, 1], fd.at[pl.ds(base, GB)], wsem)

    def wwait(i):
        slot = lax.rem(i, 2)
        base = base0 + i * GB
        pltpu.make_async_copy(rows_v.at[slot, 0],
                              fs.at[pl.ds(base, GB)], wsem).wait()
        pltpu.make_async_copy(rows_v.at[slot, 1],
                              fd.at[pl.ds(base, GB)], wsem).wait()

    start(0)

    def step(i, carry):
        @pl.when(i >= 1)
        def _():
            wwait(i - 1)

        @pl.when(i + 1 < NCHUNK)
        def _():
            start(i + 1)

        gwait(i)
        wstart(i)
        return carry

    lax.fori_loop(0, NCHUNK, step, 0)
    wwait(NCHUNK - 1)


_gather = pl.kernel(
    _gather_body,
    out_type=(jax.ShapeDtypeStruct((E, D), jnp.float32),
              jax.ShapeDtypeStruct((E, D), jnp.float32)),
    mesh=_MESH,
    scratch_types=[pltpu.VMEM((2, 2, GB), jnp.int32),
                   pltpu.VMEM((2, 2, GB, D), jnp.float32),
                   pltpu.SemaphoreType.DMA,
                   pltpu.SemaphoreType.DMA],
    compiler_params=_SC_PARAMS,
)


def _edge_body(fs_ref, fd_ref, attn_ref, s_ref, t_ref, o_ref):
    fs = fs_ref[...]
    e = fs + fd_ref[...]
    lr = jnp.where(e >= 0, e, NEG * e)
    w = lr * attn_ref[0][None, :]
    S = s_ref[...]
    z = jnp.dot(w, S, preferred_element_type=jnp.float32)
    p = jnp.exp(z)
    p128 = jnp.dot(p, S.T, preferred_element_type=jnp.float32)
    msg = fs * p128
    p16 = jnp.dot(p, t_ref[...], preferred_element_type=jnp.float32)
    o_ref[...] = jnp.concatenate([msg, p16], axis=1)


def _edge(fs, fd, attn_full, S, T, be=2000):
    return pl.pallas_call(
        _edge_body,
        grid=(E // be,),
        in_specs=[pl.BlockSpec((be, D), lambda i: (i, 0)),
                  pl.BlockSpec((be, D), lambda i: (i, 0)),
                  pl.BlockSpec((8, D), lambda i: (0, 0)),
                  pl.BlockSpec((D, 8), lambda i: (0, 0)),
                  pl.BlockSpec((8, 16), lambda i: (0, 0))],
        out_specs=pl.BlockSpec((be, WPAD), lambda i: (i, 0)),
        out_shape=jax.ShapeDtypeStruct((E, WPAD), jnp.float32),
    )(fs, fd, attn_full, S, T)


def _scatter_body(msgp, dst, out, idx_v, rows_v, zrow_v, asem, acc):
    c = lax.axis_index("c")
    s = lax.axis_index("s")
    wid = c * NS + s

    for r in range(ZB):
        for cc in range(WPAD // 16):
            zrow_v[r, pl.ds(cc * 16, 16)] = jnp.zeros((16,), jnp.float32)

    def zstep(j, carry):
        pltpu.sync_copy(zrow_v, acc.at[pl.ds(s * RPW + j * ZB, ZB)])
        return carry

    lax.fori_loop(0, RPW // ZB, zstep, 0)
    plsc.subcore_barrier()

    base0 = wid * EPW

    def stage(i):
        slot = lax.rem(i, 2)
        base = base0 + i * GB
        pltpu.sync_copy(dst.at[pl.ds(base, GB)], idx_v.at[slot])
        pltpu.sync_copy(msgp.at[pl.ds(base, GB)], rows_v.at[slot])

    def add_start(i):
        slot = lax.rem(i, 2)
        pltpu.async_copy(rows_v.at[slot], acc.at[idx_v.at[slot]], asem,
                         add=True)

    def add_wait(i):
        slot = lax.rem(i, 2)
        pltpu.make_async_copy(rows_v.at[slot], acc.at[idx_v.at[slot]],
                              asem).wait()

    stage(0)

    def step(i, carry):
        add_start(i)

        @pl.when(i >= 1)
        def _():
            add_wait(i - 1)

        @pl.when(i + 1 < NCHUNK)
        def _():
            stage(i + 1)

        return carry

    lax.fori_loop(0, NCHUNK, step, 0)
    add_wait(NCHUNK - 1)
    plsc.subcore_barrier()
    pltpu.sync_copy(acc.at[pl.ds(s * RPW, RPW)],
                    out.at[c, pl.ds(s * RPW, RPW)])


_scatter = pl.kernel(
    _scatter_body,
    out_type=jax.ShapeDtypeStruct((NC, N, WPAD), jnp.float32),
    mesh=_MESH,
    scratch_types=[pltpu.VMEM((2, GB), jnp.int32),
                   pltpu.VMEM((2, GB, WPAD), jnp.float32),
                   pltpu.VMEM((ZB, WPAD), jnp.float32),
                   pltpu.SemaphoreType.DMA,
                   pltpu.VMEM_SHARED((N, WPAD), jnp.float32)],
    compiler_params=_SC_PARAMS,
)


def _fin_mm_body(acc_ref, emat_ref, b_ref, w_ref, o_ref):
    a = acc_ref[0] + acc_ref[1]
    den = jnp.dot(a, emat_ref[...], preferred_element_type=jnp.float32)
    o = a[:, :D] / (den + 1e-9) + b_ref[0][None, :]
    o = jnp.where(o > 0, o, jnp.exp(o) - 1.0)
    o_ref[...] = jnp.dot(o, w_ref[...], preferred_element_type=jnp.float32)


def _fin_body(acc_ref, emat_ref, b_ref, o_ref):
    a = acc_ref[0] + acc_ref[1]
    den = jnp.dot(a, emat_ref[...], preferred_element_type=jnp.float32)
    o_ref[...] = a[:, :D] / (den + 1e-9) + b_ref[0][None, :]


def _finalize(acc, emat, b8, w=None, bn=2000):
    in_specs = [pl.BlockSpec((NC, bn, WPAD), lambda i: (0, i, 0)),
                pl.BlockSpec((WPAD, D), lambda i: (0, 0)),
                pl.BlockSpec((8, D), lambda i: (0, 0))]
    args = [acc, emat, b8]
    body = _fin_body
    if w is not None:
        in_specs.append(pl.BlockSpec((D, D), lambda i: (0, 0)))
        args.append(w)
        body = _fin_mm_body
    return pl.pallas_call(
        body,
        grid=(N // bn,),
        in_specs=in_specs,
        out_specs=pl.BlockSpec((bn, D), lambda i: (i, 0)),
        out_shape=jax.ShapeDtypeStruct((N, D), jnp.float32),
    )(*args)


def _head_maps(heads, hid):
    S = np.zeros((D, 8), np.float32)
    for h in range(heads):
        S[h * hid:(h + 1) * hid, h] = 1.0
    T = np.zeros((8, 16), np.float32)
    for h in range(heads):
        T[h, h] = 1.0
    emat = np.zeros((WPAD, D), np.float32)
    for h in range(heads):
        emat[D + h, h * hid:(h + 1) * hid] = 1.0
    return S, T, emat


_S0, _T0, _E0 = _head_maps(4, 32)
_S1, _T1, _E1 = _head_maps(1, 128)


def _bcast8(v):
    return jnp.broadcast_to(v.reshape(1, D), (8, D))


def kernel(h, edge_index, W0, attn0, b0, W1, attn1, b1):
    src = edge_index[0]
    dst = edge_index[1]

    def layer(feat, attn, S, T, emat, b, w_next):
        fs, fd = _gather(feat, src, dst)
        msgp = _edge(fs, fd, _bcast8(attn.reshape(-1)),
                     jnp.asarray(S), jnp.asarray(T))
        acc = _scatter(msgp, dst)
        return _finalize(acc, jnp.asarray(emat), _bcast8(b), w_next)

    feat0 = _matmul(h, W0)
    feat1 = layer(feat0, attn0, _S0, _T0, _E0, b0, W1)
    out = layer(feat1, attn1, _S1, _T1, _E1, b1, None)
    return out
